# baseline (speedup 1.0000x reference)
import jax
import jax.numpy as jnp
from jax.experimental import pallas as pl
from jax.experimental.pallas import tpu as pltpu

_EPS = 1e-12


_NSPLIT = 4


def _guidance_kernel(p_ref, *refs):
    xs = refs[:_NSPLIT]
    o_ref = refs[_NSPLIT]
    csub = p_ref.shape[1] // _NSPLIT
    s = None
    norm2 = None
    for k in range(_NSPLIT):
        xk = xs[k][0, 0]
        pk = p_ref[:, k * csub:(k + 1) * csub]
        sk = jnp.dot(pk, xk, preferred_element_type=jnp.float32)
        nk = jnp.sum(xk * xk, axis=0)
        s = sk if s is None else s + sk
        norm2 = nk if norm2 is None else norm2 + nk
    ev_f = jnp.max(s[:16], axis=0)
    ev_a = jnp.max(s[16:], axis=0)
    norm = jnp.maximum(jnp.sqrt(norm2), _EPS)
    o_ref[0] = ((ev_f - ev_a) / norm)[None, :]


def kernel(x, forgery_protos, authentic_protos):
    b, c, h, w = x.shape
    hw = h * w
    protos = jnp.concatenate([forgery_protos, authentic_protos], axis=0)
    x3 = x.reshape(b, c, hw)

    csub = c // _NSPLIT
    x4 = x.reshape(b, _NSPLIT, csub, hw)

    def _mk_spec(k):
        return pl.BlockSpec((1, 1, csub, hw), lambda i, kk=k: (i, kk, 0, 0))

    out = pl.pallas_call(
        _guidance_kernel,
        grid=(b,),
        in_specs=[pl.BlockSpec((protos.shape[0], c), lambda i: (0, 0))]
        + [_mk_spec(k) for k in range(_NSPLIT)],
        out_specs=pl.BlockSpec((1, 1, hw), lambda i: (i, 0, 0)),
        out_shape=jax.ShapeDtypeStruct((b, 1, hw), jnp.float32),
        compiler_params=pltpu.CompilerParams(
            dimension_semantics=("parallel",),
        ),
    )(protos, *([x4] * _NSPLIT))

    return out.reshape(b, 1, h, w)


# speedup vs baseline: 3.1511x; 3.1511x over previous
import jax
import jax.numpy as jnp
from jax.experimental import pallas as pl
from jax.experimental.pallas import tpu as pltpu

_EPS = 1e-12


_BBLK = 2


def _guidance_kernel(p_ref, x_ref, o_ref):
    for bi in range(_BBLK):
        xb = x_ref[bi]
        s = jnp.dot(p_ref[...], xb, preferred_element_type=jnp.float32)
        ev_f = jnp.max(s[:16], axis=0)
        ev_a = jnp.max(s[16:], axis=0)
        norm2 = jnp.sum(xb * xb, axis=0)
        norm = jnp.maximum(jnp.sqrt(norm2), _EPS)
        o_ref[bi] = ((ev_f - ev_a) / norm)[None, :]


def kernel(x, forgery_protos, authentic_protos):
    b, c, h, w = x.shape
    hw = h * w
    protos = jnp.concatenate([forgery_protos, authentic_protos], axis=0)
    x3 = x.reshape(b, c, hw)

    out = pl.pallas_call(
        _guidance_kernel,
        grid=(b // _BBLK,),
        in_specs=[
            pl.BlockSpec((protos.shape[0], c), lambda i: (0, 0)),
            pl.BlockSpec((_BBLK, c, hw), lambda i: (i, 0, 0)),
        ],
        out_specs=pl.BlockSpec((_BBLK, 1, hw), lambda i: (i, 0, 0)),
        out_shape=jax.ShapeDtypeStruct((b, 1, hw), jnp.float32),
        compiler_params=pltpu.CompilerParams(
            dimension_semantics=("parallel",),
        ),
    )(protos, x3)

    return out.reshape(b, 1, h, w)
